# speedup vs baseline: 1.1481x; 1.1481x over previous
import functools

import jax
import jax.numpy as jnp
from jax import lax
from jax.experimental import pallas as pl
from jax.experimental.pallas import tpu as pltpu
from jax.experimental.pallas import tpu_sc as plsc

B = 2
N_IN = 327680
N_OUT = 81920
D = 64

ROWS = B * N_OUT
NW = 32
ROWS_PER_W = ROWS // NW
CHUNK = 512
NCHUNK = ROWS_PER_W // CHUNK
NBUF = 2


def _make_kernel():
    mesh = plsc.VectorSubcoreMesh(core_axis_name="c", subcore_axis_name="s")

    @functools.partial(
        pl.kernel,
        mesh=mesh,
        out_type=jax.ShapeDtypeStruct((ROWS, D), jnp.float32),
        scratch_types=(
            [pltpu.VMEM((CHUNK, D), jnp.float32) for _ in range(NBUF)]
            + [pltpu.SemaphoreType.DMA for _ in range(2 * NBUF)]
        ),
    )
    def k(x_hbm, out_hbm, buf0, buf1, isem0, isem1, osem0, osem1):
        bufs = (buf0, buf1)
        isems = (isem0, isem1)
        osems = (osem0, osem1)
        wid = lax.axis_index("s") * 2 + lax.axis_index("c")
        base = wid * ROWS_PER_W

        def in_copy(ci, slot):
            off = base + ci * CHUNK
            return pltpu.make_async_copy(
                x_hbm.at[pl.ds(off, CHUNK), 3], bufs[slot], isems[slot]
            )

        def out_copy(ci, slot):
            off = base + ci * CHUNK
            return pltpu.make_async_copy(
                bufs[slot], out_hbm.at[pl.ds(off, CHUNK)], osems[slot]
            )

        for s in range(min(NBUF, NCHUNK)):
            in_copy(s, s).start()

        for ci in range(NCHUNK):
            slot = ci % NBUF
            in_copy(ci, slot).wait()
            out_copy(ci, slot).start()
            out_copy(ci, slot).wait()
            nxt = ci + NBUF
            if nxt < NCHUNK:
                in_copy(nxt, slot).start()

    return k


_sc_copy = _make_kernel()


@jax.jit
def kernel(x):
    xg = x.reshape(ROWS, 4, D)
    out = _sc_copy(xg)
    return out.reshape(B, N_OUT, D)
